# baseline (speedup 1.0000x reference)
import functools

import jax
import jax.numpy as jnp
from jax import lax
from jax.experimental import pallas as pl
from jax.experimental.pallas import tpu as pltpu, tpu_sc as plsc

GAMMA = 12.0
HIDDEN_DIM = 128
BATCH = 16384
NC, NS, L = 2, 16, 16
NW = NC * NS
PER_W = BATCH // NW
CHUNK = 64
NCHUNK = PER_W // CHUNK
SLOTS = 3


def _body(smp_hbm, ent_hbm, rel_hbm, out_hbm,
          smp_v, idx_v, hr0, hr1, hr2, tt0, tt1, tt2, res,
          sht0, sht1, sht2, sr0, sr1, sr2):
    wid = lax.axis_index("s") * NC + lax.axis_index("c")
    base = wid * PER_W
    lane = lax.iota(jnp.int32, L)
    hr = (hr0, hr1, hr2)
    tt = (tt0, tt1, tt2)
    sht = (sht0, sht1, sht2)
    sr = (sr0, sr1, sr2)

    pltpu.sync_copy(smp_hbm.at[pl.ds(base * 3, PER_W * 3)], smp_v)
    lane3 = lane * 3
    for tab in range(3):
        for c in range(NCHUNK):
            for q in range(CHUNK // L):
                j0 = c * CHUNK + q * L
                v = plsc.load_gather(smp_v, [lane3 + (3 * j0 + tab)])
                idx_v[tab, c, pl.ds(q * L, L)] = v

    def fire_ht(c):
        s = c % SLOTS
        return (pltpu.async_copy(ent_hbm.at[idx_v.at[0, c]], hr[s], sht[s]),
                pltpu.async_copy(ent_hbm.at[idx_v.at[2, c]], tt[s], sht[s]))

    def fire_r(c):
        s = c % SLOTS
        return pltpu.async_copy(rel_hbm.at[idx_v.at[1, c]], hr[s], sr[s],
                                add=True)

    DBLK = 8

    def compute(c):
        s = c % SLOTS
        rhr, rtt = hr[s], tt[s]

        def group(g, _):
            row = g * L + lane

            def dblock(db, carry):
                acc0, acc1 = carry
                dbase = db * DBLK
                for u in range(DBLK):
                    col = (lane + dbase + u) & (HIDDEN_DIM - 1)
                    a = plsc.load_gather(rhr, [row, col])
                    t = plsc.load_gather(rtt, [row, col])
                    v = jnp.abs(a - t)
                    if u % 2 == 0:
                        acc0 = acc0 + v
                    else:
                        acc1 = acc1 + v
                return acc0, acc1

            z = jnp.zeros((L,), jnp.float32)
            acc0, acc1 = lax.fori_loop(0, HIDDEN_DIM // DBLK, dblock, (z, z))
            res[pl.ds(c * CHUNK + g * L, L)] = GAMMA - (acc0 + acc1)
            return _

        lax.fori_loop(0, CHUNK // L, group, None)

    pend_ht = {0: fire_ht(0), 1: fire_ht(1)}
    pend_r = {}
    for cp in pend_ht.pop(0):
        cp.wait()
    pend_r[0] = fire_r(0)
    for c in range(NCHUNK):
        if c + 2 < NCHUNK:
            pend_ht[c + 2] = fire_ht(c + 2)
        if c + 1 < NCHUNK:
            for cp in pend_ht.pop(c + 1):
                cp.wait()
            pend_r[c + 1] = fire_r(c + 1)
        pend_r.pop(c).wait()
        compute(c)
    pltpu.sync_copy(res, out_hbm.at[pl.ds(base, PER_W)])


@jax.jit
def kernel(sample, entity_embedding, relation_embedding):
    mesh = plsc.VectorSubcoreMesh(core_axis_name="c", subcore_axis_name="s",
                                  num_cores=NC, num_subcores=NS)
    run = pl.kernel(
        _body,
        out_type=jax.ShapeDtypeStruct((BATCH,), jnp.float32),
        mesh=mesh,
        compiler_params=pltpu.CompilerParams(needs_layout_passes=False),
        scratch_types=[
            pltpu.VMEM((PER_W * 3,), jnp.int32),
            pltpu.VMEM((3, NCHUNK, CHUNK), jnp.int32),
            pltpu.VMEM((CHUNK, HIDDEN_DIM), jnp.float32),
            pltpu.VMEM((CHUNK, HIDDEN_DIM), jnp.float32),
            pltpu.VMEM((CHUNK, HIDDEN_DIM), jnp.float32),
            pltpu.VMEM((CHUNK, HIDDEN_DIM), jnp.float32),
            pltpu.VMEM((CHUNK, HIDDEN_DIM), jnp.float32),
            pltpu.VMEM((CHUNK, HIDDEN_DIM), jnp.float32),
            pltpu.VMEM((PER_W,), jnp.float32),
            pltpu.SemaphoreType.DMA,
            pltpu.SemaphoreType.DMA,
            pltpu.SemaphoreType.DMA,
            pltpu.SemaphoreType.DMA,
            pltpu.SemaphoreType.DMA,
            pltpu.SemaphoreType.DMA,
        ],
    )
    score = run(sample.astype(jnp.int32).reshape(-1), entity_embedding,
                relation_embedding)
    return score[:, None]


# speedup vs baseline: 1.3806x; 1.3806x over previous
import functools

import jax
import jax.numpy as jnp
from jax import lax
from jax.experimental import pallas as pl
from jax.experimental.pallas import tpu as pltpu, tpu_sc as plsc

GAMMA = 12.0
HIDDEN_DIM = 128
BATCH = 16384
NC, NS, L = 2, 16, 16
NW = NC * NS
PER_W = BATCH // NW
CHUNK = 64
NCHUNK = PER_W // CHUNK
SLOTS = 3


def _body(idx_hbm, ent_hbm, rel_hbm, out_hbm,
          idx_v, hh0, hh1, hh2, rr0, rr1, rr2, tt0, tt1, tt2, res,
          sem0, sem1, sem2):
    wid = lax.axis_index("s") * NC + lax.axis_index("c")
    base = wid * PER_W
    lane = lax.iota(jnp.int32, L)
    hh = (hh0, hh1, hh2)
    rr = (rr0, rr1, rr2)
    tt = (tt0, tt1, tt2)
    sems = (sem0, sem1, sem2)

    pltpu.sync_copy(idx_hbm.at[wid], idx_v)

    def fire(c):
        s = c % SLOTS
        return (pltpu.async_copy(ent_hbm.at[idx_v.at[0, c]], hh[s], sems[s]),
                pltpu.async_copy(rel_hbm.at[idx_v.at[1, c]], rr[s], sems[s]),
                pltpu.async_copy(ent_hbm.at[idx_v.at[2, c]], tt[s], sems[s]))

    DBLK = 8

    def compute(c):
        s = c % SLOTS
        rhh, rrr, rtt = hh[s], rr[s], tt[s]

        def group(g, _):
            row = g * L + lane

            def dblock(db, carry):
                acc0, acc1 = carry
                dbase = db * DBLK
                for u in range(DBLK):
                    col = (lane + dbase + u) & (HIDDEN_DIM - 1)
                    h = plsc.load_gather(rhh, [row, col])
                    r = plsc.load_gather(rrr, [row, col])
                    t = plsc.load_gather(rtt, [row, col])
                    v = jnp.abs(h + r - t)
                    if u % 2 == 0:
                        acc0 = acc0 + v
                    else:
                        acc1 = acc1 + v
                return acc0, acc1

            z = jnp.zeros((L,), jnp.float32)
            acc0, acc1 = lax.fori_loop(0, HIDDEN_DIM // DBLK, dblock, (z, z))
            res[pl.ds(c * CHUNK + g * L, L)] = GAMMA - (acc0 + acc1)
            return _

        lax.fori_loop(0, CHUNK // L, group, None)

    pend = {0: fire(0), 1: fire(1)}
    for c in range(NCHUNK):
        if c + 2 < NCHUNK:
            pend[c + 2] = fire(c + 2)
        for cp in pend.pop(c):
            cp.wait()
        compute(c)
    pltpu.sync_copy(res, out_hbm.at[pl.ds(base, PER_W)])


@jax.jit
def kernel(sample, entity_embedding, relation_embedding):
    idx = sample.astype(jnp.int32).T.reshape(3, NW, NCHUNK, CHUNK)
    idx = jnp.swapaxes(idx, 0, 1)
    mesh = plsc.VectorSubcoreMesh(core_axis_name="c", subcore_axis_name="s",
                                  num_cores=NC, num_subcores=NS)
    run = pl.kernel(
        _body,
        out_type=jax.ShapeDtypeStruct((BATCH,), jnp.float32),
        mesh=mesh,
        compiler_params=pltpu.CompilerParams(needs_layout_passes=False),
        scratch_types=[
            pltpu.VMEM((3, NCHUNK, CHUNK), jnp.int32),
            pltpu.VMEM((CHUNK, HIDDEN_DIM), jnp.float32),
            pltpu.VMEM((CHUNK, HIDDEN_DIM), jnp.float32),
            pltpu.VMEM((CHUNK, HIDDEN_DIM), jnp.float32),
            pltpu.VMEM((CHUNK, HIDDEN_DIM), jnp.float32),
            pltpu.VMEM((CHUNK, HIDDEN_DIM), jnp.float32),
            pltpu.VMEM((CHUNK, HIDDEN_DIM), jnp.float32),
            pltpu.VMEM((CHUNK, HIDDEN_DIM), jnp.float32),
            pltpu.VMEM((CHUNK, HIDDEN_DIM), jnp.float32),
            pltpu.VMEM((CHUNK, HIDDEN_DIM), jnp.float32),
            pltpu.VMEM((PER_W,), jnp.float32),
            pltpu.SemaphoreType.DMA,
            pltpu.SemaphoreType.DMA,
            pltpu.SemaphoreType.DMA,
        ],
    )
    score = run(idx, entity_embedding, relation_embedding)
    return score[:, None]
